# Initial kernel scaffold; baseline (speedup 1.0000x reference)
import jax
import jax.numpy as jnp
from jax.experimental import pallas as pl


def kernel(x, edge_index, edge_attr, W1, b1, W2, b2, root, conv_bias, gamma, beta):
    raise NotImplementedError("write your pallas kernel here")


# speedup vs baseline: 2.0932x; 2.0932x over previous
import functools

import jax
import jax.numpy as jnp
import numpy as np
from jax import lax
from jax.experimental import pallas as pl
from jax.experimental.pallas import tpu as pltpu
from jax.experimental.pallas import tpu_sc as plsc

N = 10000
E = 160000
D_IN = 32
D_OUT = 32
D_EDGE = 16
HID = 1024
W48 = 48

NC = 2
NS = 16
NW = NC * NS

E_PAD = 163840
ROWS_W = E_PAD // NW
CHUNKS_W = ROWS_W // 128
ACC_ROWS = N + 8

def _mesh():
    return plsc.VectorSubcoreMesh(core_axis_name="c", subcore_axis_name="s",
                                  num_cores=NC, num_subcores=NS)


def _gather_body(x_hbm, srcm_hbm, out_hbm, idx_v, row_v, sem):
    wid = lax.axis_index("s") * NC + lax.axis_index("c")
    pltpu.sync_copy(srcm_hbm.at[pl.ds(wid * CHUNKS_W, CHUNKS_W)], idx_v)

    def body(j, carry):
        pltpu.async_copy(x_hbm.at[idx_v.at[j]], row_v, sem).wait()
        pltpu.sync_copy(row_v, out_hbm.at[pl.ds(wid * ROWS_W + j * 128, 128)])
        return carry

    lax.fori_loop(0, CHUNKS_W, body, 0)


@jax.jit
def _sc_gather(x, src_m):
    return pl.kernel(
        _gather_body,
        out_type=jax.ShapeDtypeStruct((E_PAD, D_IN), jnp.float32),
        mesh=_mesh(),
        scratch_types=[
            pltpu.VMEM((CHUNKS_W, 128), jnp.int32),
            pltpu.VMEM((128, D_IN), jnp.float32),
            pltpu.SemaphoreType.DMA,
        ],
        compiler_params=pltpu.CompilerParams(use_tc_tiling_on_sc=False),
    )(x, src_m)


def _scatter_body(msg_hbm, dstm_hbm, zero_hbm, out0_hbm, out1_hbm,
                  idx_v, msg_v, acc_sh):
    cid = lax.axis_index("c")
    sid = lax.axis_index("s")
    wid = sid * NC + cid

    @pl.when(sid == 0)
    def _():
        pltpu.sync_copy(zero_hbm, acc_sh)

    plsc.subcore_barrier()

    pltpu.sync_copy(dstm_hbm.at[pl.ds(wid * CHUNKS_W, CHUNKS_W)], idx_v)

    def outer(c, carry):
        pltpu.sync_copy(msg_hbm.at[pl.ds(wid * ROWS_W + c * 1024, 1024)], msg_v)

        def inner(j, carry2):
            pltpu.sync_copy(msg_v.at[pl.ds(j * 128, 128)],
                            acc_sh.at[idx_v.at[c * 8 + j]], add=True)
            return carry2

        lax.fori_loop(0, 8, inner, 0)
        return carry

    lax.fori_loop(0, ROWS_W // 1024, outer, 0)
    plsc.subcore_barrier()

    rows0 = sid * (N // NS)

    @pl.when(cid == 0)
    def _():
        pltpu.sync_copy(acc_sh.at[pl.ds(rows0, N // NS)],
                        out0_hbm.at[pl.ds(rows0, N // NS)])

    @pl.when(cid == 1)
    def _():
        pltpu.sync_copy(acc_sh.at[pl.ds(rows0, N // NS)],
                        out1_hbm.at[pl.ds(rows0, N // NS)])


@jax.jit
def _sc_scatter(msg, dst_m, zeros_acc):
    return pl.kernel(
        _scatter_body,
        out_type=(jax.ShapeDtypeStruct((N, W48), jnp.float32),
                  jax.ShapeDtypeStruct((N, W48), jnp.float32)),
        mesh=_mesh(),
        scratch_types=[
            pltpu.VMEM((CHUNKS_W, 128), jnp.int32),
            pltpu.VMEM((1024, W48), jnp.float32),
            pltpu.VMEM_SHARED((ACC_ROWS, W48), jnp.float32),
        ],
        compiler_params=pltpu.CompilerParams(use_tc_tiling_on_sc=False),
    )(msg, dst_m, zeros_acc)


TE = 2048


def _edge_body(ea_ref, xj_ref, w1_ref, b1_ref, w2_ref, b2_ref,
               r_ref, s_ref, c_ref, out_ref):
    h = jnp.dot(ea_ref[...], w1_ref[...],
                preferred_element_type=jnp.float32) + b1_ref[...]
    h = jnp.where(h >= 0.0, h, 0.01 * h)
    w = jnp.dot(h, w2_ref[...], preferred_element_type=jnp.float32) + b2_ref[...]
    xr = jnp.dot(xj_ref[...], r_ref[...], preferred_element_type=jnp.float32)
    out_ref[...] = jnp.dot(xr * w, s_ref[...],
                           preferred_element_type=jnp.float32) + c_ref[...]


_R_EXPAND = np.zeros((D_IN, HID), dtype=np.float32)
for _i in range(D_IN):
    _R_EXPAND[_i, _i * D_OUT:(_i + 1) * D_OUT] = 1.0
_S_SELECT = np.zeros((HID, W48), dtype=np.float32)
for _i in range(D_IN):
    for _o in range(D_OUT):
        _S_SELECT[_i * D_OUT + _o, _o] = 1.0
_C_ONES = np.zeros((1, W48), dtype=np.float32)
_C_ONES[0, D_OUT] = 1.0


@jax.jit
def _tc_edge(ea_p, xj, W1, b1, W2, b2):
    grid = (E_PAD // TE,)
    return pl.pallas_call(
        _edge_body,
        grid=grid,
        in_specs=[
            pl.BlockSpec((TE, D_EDGE), lambda i: (i, 0)),
            pl.BlockSpec((TE, D_IN), lambda i: (i, 0)),
            pl.BlockSpec((D_EDGE, HID), lambda i: (0, 0)),
            pl.BlockSpec((1, HID), lambda i: (0, 0)),
            pl.BlockSpec((HID, HID), lambda i: (0, 0)),
            pl.BlockSpec((1, HID), lambda i: (0, 0)),
            pl.BlockSpec((D_IN, HID), lambda i: (0, 0)),
            pl.BlockSpec((HID, W48), lambda i: (0, 0)),
            pl.BlockSpec((1, W48), lambda i: (0, 0)),
        ],
        out_specs=pl.BlockSpec((TE, W48), lambda i: (i, 0)),
        out_shape=jax.ShapeDtypeStruct((E_PAD, W48), jnp.float32),
        compiler_params=pltpu.CompilerParams(
            dimension_semantics=("arbitrary",)),
    )(ea_p, xj, W1, b1.reshape(1, HID), W2, b2.reshape(1, HID),
      jnp.asarray(_R_EXPAND), jnp.asarray(_S_SELECT), jnp.asarray(_C_ONES))


def _finish_body(p0_ref, p1_ref, x_ref, root_ref, cb_ref, g_ref, b_ref,
                 out_ref):
    p = p0_ref[...] + p1_ref[...]
    agg = p[:, :D_OUT] / jnp.maximum(p[:, D_OUT:D_OUT + 1], 1.0)
    pre = agg + jnp.dot(x_ref[...], root_ref[...],
                        preferred_element_type=jnp.float32) + cb_ref[...]
    mean = jnp.mean(pre, axis=0, keepdims=True)
    cen = pre - mean
    var = jnp.mean(cen * cen, axis=0, keepdims=True)
    out_ref[...] = cen * lax.rsqrt(var + 1e-5) * g_ref[...] + b_ref[...]


@jax.jit
def _tc_finish(p0, p1, x, root, conv_bias, gamma, beta):
    return pl.pallas_call(
        _finish_body,
        out_shape=jax.ShapeDtypeStruct((N, D_OUT), jnp.float32),
    )(p0, p1, x, root, conv_bias.reshape(1, D_OUT),
      gamma.reshape(1, D_OUT), beta.reshape(1, D_OUT))


def kernel(x, edge_index, edge_attr, W1, b1, W2, b2, root, conv_bias,
           gamma, beta):
    pad = E_PAD - E
    src = edge_index[0].astype(jnp.int32)
    dst = edge_index[1].astype(jnp.int32)
    src_m = jnp.concatenate([src, jnp.zeros((pad,), jnp.int32)]
                            ).reshape(E_PAD // 128, 128)
    dst_m = jnp.concatenate([dst, jnp.full((pad,), N, jnp.int32)]
                            ).reshape(E_PAD // 128, 128)
    ea_p = jnp.concatenate(
        [edge_attr, jnp.zeros((pad, D_EDGE), jnp.float32)], axis=0)

    xj = _sc_gather(x, src_m)
    msg = _tc_edge(ea_p, xj, W1, b1, W2, b2)
    zeros_acc = jnp.zeros((ACC_ROWS, W48), jnp.float32)
    p0, p1 = _sc_scatter(msg, dst_m, zeros_acc)
    return _tc_finish(p0, p1, x, root, conv_bias, gamma, beta)


# speedup vs baseline: 2.1018x; 1.0041x over previous
import functools

import jax
import jax.numpy as jnp
import numpy as np
from jax import lax
from jax.experimental import pallas as pl
from jax.experimental.pallas import tpu as pltpu
from jax.experimental.pallas import tpu_sc as plsc

N = 10000
E = 160000
D_IN = 32
D_OUT = 32
D_EDGE = 16
HID = 1024
W48 = 48

NC = 2
NS = 16
NW = NC * NS

E_PAD = 163840
ROWS_W = E_PAD // NW
CHUNKS_W = ROWS_W // 128
ACC_ROWS = N + 8

def _mesh():
    return plsc.VectorSubcoreMesh(core_axis_name="c", subcore_axis_name="s",
                                  num_cores=NC, num_subcores=NS)


def _gather_body(x_hbm, srcm_hbm, out_hbm, idx_v, row_v, sem):
    wid = lax.axis_index("s") * NC + lax.axis_index("c")
    pltpu.sync_copy(srcm_hbm.at[pl.ds(wid * CHUNKS_W, CHUNKS_W)], idx_v)

    def body(j, carry):
        pltpu.async_copy(x_hbm.at[idx_v.at[j]], row_v, sem).wait()
        pltpu.sync_copy(row_v, out_hbm.at[pl.ds(wid * ROWS_W + j * 128, 128)])
        return carry

    lax.fori_loop(0, CHUNKS_W, body, 0)


@jax.jit
def _sc_gather(x, src_m):
    return pl.kernel(
        _gather_body,
        out_type=jax.ShapeDtypeStruct((E_PAD, D_IN), jnp.float32),
        mesh=_mesh(),
        scratch_types=[
            pltpu.VMEM((CHUNKS_W, 128), jnp.int32),
            pltpu.VMEM((128, D_IN), jnp.float32),
            pltpu.SemaphoreType.DMA,
        ],
        compiler_params=pltpu.CompilerParams(use_tc_tiling_on_sc=False),
    )(x, src_m)


def _scatter_body(msg_hbm, dstm_hbm, zero_hbm, out0_hbm, out1_hbm,
                  idx_v, msg_v, acc_sh):
    cid = lax.axis_index("c")
    sid = lax.axis_index("s")
    wid = sid * NC + cid

    @pl.when(sid == 0)
    def _():
        pltpu.sync_copy(zero_hbm, acc_sh)

    plsc.subcore_barrier()

    pltpu.sync_copy(dstm_hbm.at[pl.ds(wid * CHUNKS_W, CHUNKS_W)], idx_v)

    def outer(c, carry):
        pltpu.sync_copy(msg_hbm.at[pl.ds(wid * ROWS_W + c * 1024, 1024)], msg_v)

        def inner(j, carry2):
            pltpu.sync_copy(msg_v.at[pl.ds(j * 128, 128)],
                            acc_sh.at[idx_v.at[c * 8 + j]], add=True)
            return carry2

        lax.fori_loop(0, 8, inner, 0)
        return carry

    lax.fori_loop(0, ROWS_W // 1024, outer, 0)
    plsc.subcore_barrier()

    rows0 = sid * (N // NS)

    @pl.when(cid == 0)
    def _():
        pltpu.sync_copy(acc_sh.at[pl.ds(rows0, N // NS)],
                        out0_hbm.at[pl.ds(rows0, N // NS)])

    @pl.when(cid == 1)
    def _():
        pltpu.sync_copy(acc_sh.at[pl.ds(rows0, N // NS)],
                        out1_hbm.at[pl.ds(rows0, N // NS)])


@jax.jit
def _sc_scatter(msg, dst_m, zeros_acc):
    return pl.kernel(
        _scatter_body,
        out_type=(jax.ShapeDtypeStruct((N, W48), jnp.float32),
                  jax.ShapeDtypeStruct((N, W48), jnp.float32)),
        mesh=_mesh(),
        scratch_types=[
            pltpu.VMEM((CHUNKS_W, 128), jnp.int32),
            pltpu.VMEM((1024, W48), jnp.float32),
            pltpu.VMEM_SHARED((ACC_ROWS, W48), jnp.float32),
        ],
        compiler_params=pltpu.CompilerParams(use_tc_tiling_on_sc=False),
    )(msg, dst_m, zeros_acc)


TE = 2048


def _edge_body(ea_ref, xj_ref, w1_ref, b1_ref, w2_ref, b2_ref,
               r_ref, s_ref, c_ref, out_ref):
    h = jnp.dot(ea_ref[...], w1_ref[...],
                preferred_element_type=jnp.float32) + b1_ref[...]
    h = jnp.where(h >= 0.0, h, 0.01 * h)
    w = jnp.dot(h.astype(jnp.bfloat16), w2_ref[...],
                preferred_element_type=jnp.float32) + b2_ref[...]
    xr = jnp.dot(xj_ref[...], r_ref[...], preferred_element_type=jnp.float32)
    out_ref[...] = jnp.dot(xr * w, s_ref[...],
                           preferred_element_type=jnp.float32) + c_ref[...]


_R_EXPAND = np.zeros((D_IN, HID), dtype=np.float32)
for _i in range(D_IN):
    _R_EXPAND[_i, _i * D_OUT:(_i + 1) * D_OUT] = 1.0
_S_SELECT = np.zeros((HID, W48), dtype=np.float32)
for _i in range(D_IN):
    for _o in range(D_OUT):
        _S_SELECT[_i * D_OUT + _o, _o] = 1.0
_C_ONES = np.zeros((1, W48), dtype=np.float32)
_C_ONES[0, D_OUT] = 1.0


@jax.jit
def _tc_edge(ea_p, xj, W1, b1, W2, b2):
    grid = (E_PAD // TE,)
    return pl.pallas_call(
        _edge_body,
        grid=grid,
        in_specs=[
            pl.BlockSpec((TE, D_EDGE), lambda i: (i, 0)),
            pl.BlockSpec((TE, D_IN), lambda i: (i, 0)),
            pl.BlockSpec((D_EDGE, HID), lambda i: (0, 0)),
            pl.BlockSpec((1, HID), lambda i: (0, 0)),
            pl.BlockSpec((HID, HID), lambda i: (0, 0)),
            pl.BlockSpec((1, HID), lambda i: (0, 0)),
            pl.BlockSpec((D_IN, HID), lambda i: (0, 0)),
            pl.BlockSpec((HID, W48), lambda i: (0, 0)),
            pl.BlockSpec((1, W48), lambda i: (0, 0)),
        ],
        out_specs=pl.BlockSpec((TE, W48), lambda i: (i, 0)),
        out_shape=jax.ShapeDtypeStruct((E_PAD, W48), jnp.float32),
        compiler_params=pltpu.CompilerParams(
            dimension_semantics=("arbitrary",)),
    )(ea_p, xj, W1, b1.reshape(1, HID), W2.astype(jnp.bfloat16),
      b2.reshape(1, HID),
      jnp.asarray(_R_EXPAND), jnp.asarray(_S_SELECT), jnp.asarray(_C_ONES))


def _finish_body(p0_ref, p1_ref, x_ref, root_ref, cb_ref, g_ref, b_ref,
                 out_ref):
    p = p0_ref[...] + p1_ref[...]
    agg = p[:, :D_OUT] / jnp.maximum(p[:, D_OUT:D_OUT + 1], 1.0)
    pre = agg + jnp.dot(x_ref[...], root_ref[...],
                        preferred_element_type=jnp.float32) + cb_ref[...]
    mean = jnp.mean(pre, axis=0, keepdims=True)
    cen = pre - mean
    var = jnp.mean(cen * cen, axis=0, keepdims=True)
    out_ref[...] = cen * lax.rsqrt(var + 1e-5) * g_ref[...] + b_ref[...]


@jax.jit
def _tc_finish(p0, p1, x, root, conv_bias, gamma, beta):
    return pl.pallas_call(
        _finish_body,
        out_shape=jax.ShapeDtypeStruct((N, D_OUT), jnp.float32),
    )(p0, p1, x, root, conv_bias.reshape(1, D_OUT),
      gamma.reshape(1, D_OUT), beta.reshape(1, D_OUT))


def kernel(x, edge_index, edge_attr, W1, b1, W2, b2, root, conv_bias,
           gamma, beta):
    pad = E_PAD - E
    src = edge_index[0].astype(jnp.int32)
    dst = edge_index[1].astype(jnp.int32)
    src_m = jnp.concatenate([src, jnp.zeros((pad,), jnp.int32)]
                            ).reshape(E_PAD // 128, 128)
    dst_m = jnp.concatenate([dst, jnp.full((pad,), N, jnp.int32)]
                            ).reshape(E_PAD // 128, 128)
    ea_p = jnp.concatenate(
        [edge_attr, jnp.zeros((pad, D_EDGE), jnp.float32)], axis=0)

    xj = _sc_gather(x, src_m)
    msg = _tc_edge(ea_p, xj, W1, b1, W2, b2)
    zeros_acc = jnp.zeros((ACC_ROWS, W48), jnp.float32)
    p0, p1 = _sc_scatter(msg, dst_m, zeros_acc)
    return _tc_finish(p0, p1, x, root, conv_bias, gamma, beta)


# speedup vs baseline: 2.3051x; 1.0967x over previous
import jax
import jax.numpy as jnp
import numpy as np
from jax import lax
from jax.experimental import pallas as pl
from jax.experimental.pallas import tpu as pltpu
from jax.experimental.pallas import tpu_sc as plsc

N = 10000
E = 160000
D_IN = 32
D_OUT = 32
D_EDGE = 16
HID = 1024
W48 = 48

NC = 2
NS = 16
NW = NC * NS

ROWS_W = E // NW
CHUNK = 125
CHUNKS_W = ROWS_W // CHUNK
OUTER = 5
INNER = CHUNKS_W // OUTER


def _mesh():
    return plsc.VectorSubcoreMesh(core_axis_name="c", subcore_axis_name="s",
                                  num_cores=NC, num_subcores=NS)


def _gather_body(x_hbm, srcm_hbm, out_hbm, idx_v, row_v, sem):
    wid = lax.axis_index("s") * NC + lax.axis_index("c")
    pltpu.sync_copy(srcm_hbm.at[pl.ds(wid * CHUNKS_W, CHUNKS_W)], idx_v)

    def body(j, carry):
        pltpu.async_copy(x_hbm.at[idx_v.at[j]], row_v, sem).wait()
        pltpu.sync_copy(row_v,
                        out_hbm.at[pl.ds(wid * ROWS_W + j * CHUNK, CHUNK)])
        return carry

    lax.fori_loop(0, CHUNKS_W, body, 0)


@jax.jit
def _sc_gather(x, src_m):
    return pl.kernel(
        _gather_body,
        out_type=jax.ShapeDtypeStruct((E, D_IN), jnp.float32),
        mesh=_mesh(),
        scratch_types=[
            pltpu.VMEM((CHUNKS_W, CHUNK), jnp.int32),
            pltpu.VMEM((CHUNK, D_IN), jnp.float32),
            pltpu.SemaphoreType.DMA,
        ],
        compiler_params=pltpu.CompilerParams(use_tc_tiling_on_sc=False),
    )(x, src_m)


def _scatter_body(msg_hbm, dstm_hbm, zero_hbm, out0_hbm, out1_hbm,
                  idx_v, msg_v, acc_sh):
    cid = lax.axis_index("c")
    sid = lax.axis_index("s")
    wid = sid * NC + cid

    @pl.when(sid == 0)
    def _():
        pltpu.sync_copy(zero_hbm, acc_sh)

    plsc.subcore_barrier()

    pltpu.sync_copy(dstm_hbm.at[pl.ds(wid * CHUNKS_W, CHUNKS_W)], idx_v)

    def outer(c, carry):
        pltpu.sync_copy(
            msg_hbm.at[pl.ds(wid * ROWS_W + c * (INNER * CHUNK),
                             INNER * CHUNK)], msg_v)

        def inner(j, carry2):
            pltpu.sync_copy(msg_v.at[pl.ds(j * CHUNK, CHUNK)],
                            acc_sh.at[idx_v.at[c * INNER + j]], add=True)
            return carry2

        lax.fori_loop(0, INNER, inner, 0)
        return carry

    lax.fori_loop(0, OUTER, outer, 0)
    plsc.subcore_barrier()

    rows0 = sid * (N // NS)

    @pl.when(cid == 0)
    def _():
        pltpu.sync_copy(acc_sh.at[pl.ds(rows0, N // NS)],
                        out0_hbm.at[pl.ds(rows0, N // NS)])

    @pl.when(cid == 1)
    def _():
        pltpu.sync_copy(acc_sh.at[pl.ds(rows0, N // NS)],
                        out1_hbm.at[pl.ds(rows0, N // NS)])


@jax.jit
def _sc_scatter(msg, dst_m, zeros_acc):
    return pl.kernel(
        _scatter_body,
        out_type=(jax.ShapeDtypeStruct((N, W48), jnp.float32),
                  jax.ShapeDtypeStruct((N, W48), jnp.float32)),
        mesh=_mesh(),
        scratch_types=[
            pltpu.VMEM((CHUNKS_W, CHUNK), jnp.int32),
            pltpu.VMEM((INNER * CHUNK, W48), jnp.float32),
            pltpu.VMEM_SHARED((N, W48), jnp.float32),
        ],
        compiler_params=pltpu.CompilerParams(use_tc_tiling_on_sc=False),
    )(msg, dst_m, zeros_acc)


TE = 2000


def _edge_body(ea_ref, xj_ref, w1_ref, b1_ref, w2_ref, b2_ref,
               r_ref, s_ref, c_ref, out_ref):
    h = jnp.dot(ea_ref[...].astype(jnp.bfloat16), w1_ref[...],
                preferred_element_type=jnp.float32) + b1_ref[...]
    h = jnp.where(h >= 0.0, h, 0.01 * h)
    w = jnp.dot(h.astype(jnp.bfloat16), w2_ref[...],
                preferred_element_type=jnp.float32) + b2_ref[...]
    xr = jnp.dot(xj_ref[...].astype(jnp.bfloat16), r_ref[...],
                 preferred_element_type=jnp.float32)
    prod = (xr * w).astype(jnp.bfloat16)
    out_ref[...] = jnp.dot(prod, s_ref[...],
                           preferred_element_type=jnp.float32) + c_ref[...]


_R_EXPAND = np.zeros((D_IN, HID), dtype=np.float32)
for _i in range(D_IN):
    _R_EXPAND[_i, _i * D_OUT:(_i + 1) * D_OUT] = 1.0
_S_SELECT = np.zeros((HID, W48), dtype=np.float32)
for _i in range(D_IN):
    for _o in range(D_OUT):
        _S_SELECT[_i * D_OUT + _o, _o] = 1.0
_C_ONES = np.zeros((1, W48), dtype=np.float32)
_C_ONES[0, D_OUT] = 1.0


@jax.jit
def _tc_edge(ea, xj, W1, b1, W2, b2):
    grid = (E // TE,)
    return pl.pallas_call(
        _edge_body,
        grid=grid,
        in_specs=[
            pl.BlockSpec((TE, D_EDGE), lambda i: (i, 0)),
            pl.BlockSpec((TE, D_IN), lambda i: (i, 0)),
            pl.BlockSpec((D_EDGE, HID), lambda i: (0, 0)),
            pl.BlockSpec((1, HID), lambda i: (0, 0)),
            pl.BlockSpec((HID, HID), lambda i: (0, 0)),
            pl.BlockSpec((1, HID), lambda i: (0, 0)),
            pl.BlockSpec((D_IN, HID), lambda i: (0, 0)),
            pl.BlockSpec((HID, W48), lambda i: (0, 0)),
            pl.BlockSpec((1, W48), lambda i: (0, 0)),
        ],
        out_specs=pl.BlockSpec((TE, W48), lambda i: (i, 0)),
        out_shape=jax.ShapeDtypeStruct((E, W48), jnp.float32),
        compiler_params=pltpu.CompilerParams(
            dimension_semantics=("arbitrary",)),
    )(ea, xj, W1.astype(jnp.bfloat16), b1.reshape(1, HID),
      W2.astype(jnp.bfloat16), b2.reshape(1, HID),
      jnp.asarray(_R_EXPAND, dtype=jnp.bfloat16),
      jnp.asarray(_S_SELECT, dtype=jnp.bfloat16), jnp.asarray(_C_ONES))


def _finish_body(p0_ref, p1_ref, x_ref, root_ref, cb_ref, g_ref, b_ref,
                 out_ref):
    p = p0_ref[...] + p1_ref[...]
    agg = p[:, :D_OUT] / jnp.maximum(p[:, D_OUT:D_OUT + 1], 1.0)
    pre = agg + jnp.dot(x_ref[...], root_ref[...],
                        preferred_element_type=jnp.float32) + cb_ref[...]
    mean = jnp.mean(pre, axis=0, keepdims=True)
    cen = pre - mean
    var = jnp.mean(cen * cen, axis=0, keepdims=True)
    out_ref[...] = cen * lax.rsqrt(var + 1e-5) * g_ref[...] + b_ref[...]


@jax.jit
def _tc_finish(p0, p1, x, root, conv_bias, gamma, beta):
    return pl.pallas_call(
        _finish_body,
        out_shape=jax.ShapeDtypeStruct((N, D_OUT), jnp.float32),
    )(p0, p1, x, root, conv_bias.reshape(1, D_OUT),
      gamma.reshape(1, D_OUT), beta.reshape(1, D_OUT))


def kernel(x, edge_index, edge_attr, W1, b1, W2, b2, root, conv_bias,
           gamma, beta):
    src_m = edge_index[0].astype(jnp.int32).reshape(NW * CHUNKS_W, CHUNK)
    dst_m = edge_index[1].astype(jnp.int32).reshape(NW * CHUNKS_W, CHUNK)

    xj = _sc_gather(x, src_m)
    msg = _tc_edge(edge_attr, xj, W1, b1, W2, b2)
    zeros_acc = jnp.zeros((N, W48), jnp.float32)
    p0, p1 = _sc_scatter(msg, dst_m, zeros_acc)
    return _tc_finish(p0, p1, x, root, conv_bias, gamma, beta)


# speedup vs baseline: 2.3861x; 1.0351x over previous
import jax
import jax.numpy as jnp
import numpy as np
from jax import lax
from jax.experimental import pallas as pl
from jax.experimental.pallas import tpu as pltpu
from jax.experimental.pallas import tpu_sc as plsc

N = 10000
E = 160000
D_IN = 32
D_OUT = 32
D_EDGE = 16
HID = 1024
W48 = 48

NC = 2
NS = 16
NW = NC * NS

ROWS_W = E // NW
CHUNK = 125
CHUNKS_W = ROWS_W // CHUNK
OUTER = 5
INNER = CHUNKS_W // OUTER


def _mesh():
    return plsc.VectorSubcoreMesh(core_axis_name="c", subcore_axis_name="s",
                                  num_cores=NC, num_subcores=NS)


def _gather_body(x_hbm, srcm_hbm, out_hbm, idx_v, row_v, sem):
    wid = lax.axis_index("s") * NC + lax.axis_index("c")
    pltpu.sync_copy(srcm_hbm.at[pl.ds(wid * CHUNKS_W, CHUNKS_W)], idx_v)

    def body(j, carry):
        pltpu.async_copy(x_hbm.at[idx_v.at[j]], row_v, sem).wait()
        pltpu.sync_copy(row_v,
                        out_hbm.at[pl.ds(wid * ROWS_W + j * CHUNK, CHUNK)])
        return carry

    lax.fori_loop(0, CHUNKS_W, body, 0)


@jax.jit
def _sc_gather(x, src_m):
    return pl.kernel(
        _gather_body,
        out_type=jax.ShapeDtypeStruct((E, D_IN), jnp.float32),
        mesh=_mesh(),
        scratch_types=[
            pltpu.VMEM((CHUNKS_W, CHUNK), jnp.int32),
            pltpu.VMEM((CHUNK, D_IN), jnp.float32),
            pltpu.SemaphoreType.DMA,
        ],
        compiler_params=pltpu.CompilerParams(use_tc_tiling_on_sc=False),
    )(x, src_m)


def _scatter_body(msg_hbm, dstm_hbm, zero_hbm, out0_hbm, out1_hbm,
                  idx_v, msg_v, acc_sh):
    cid = lax.axis_index("c")
    sid = lax.axis_index("s")
    wid = sid * NC + cid

    @pl.when(sid == 0)
    def _():
        pltpu.sync_copy(zero_hbm, acc_sh)

    plsc.subcore_barrier()

    pltpu.sync_copy(dstm_hbm.at[pl.ds(wid * CHUNKS_W, CHUNKS_W)], idx_v)

    def outer(c, carry):
        pltpu.sync_copy(
            msg_hbm.at[pl.ds(wid * ROWS_W + c * (INNER * CHUNK),
                             INNER * CHUNK)], msg_v)

        def inner(j, carry2):
            pltpu.sync_copy(msg_v.at[pl.ds(j * CHUNK, CHUNK)],
                            acc_sh.at[idx_v.at[c * INNER + j]], add=True)
            return carry2

        lax.fori_loop(0, INNER, inner, 0)
        return carry

    lax.fori_loop(0, OUTER, outer, 0)
    plsc.subcore_barrier()

    rows0 = sid * (N // NS)

    @pl.when(cid == 0)
    def _():
        pltpu.sync_copy(acc_sh.at[pl.ds(rows0, N // NS)],
                        out0_hbm.at[pl.ds(rows0, N // NS)])

    @pl.when(cid == 1)
    def _():
        pltpu.sync_copy(acc_sh.at[pl.ds(rows0, N // NS)],
                        out1_hbm.at[pl.ds(rows0, N // NS)])


@jax.jit
def _sc_scatter(msg, dst_m, zeros_acc):
    return pl.kernel(
        _scatter_body,
        out_type=(jax.ShapeDtypeStruct((N, W48), jnp.float32),
                  jax.ShapeDtypeStruct((N, W48), jnp.float32)),
        mesh=_mesh(),
        scratch_types=[
            pltpu.VMEM((CHUNKS_W, CHUNK), jnp.int32),
            pltpu.VMEM((INNER * CHUNK, W48), jnp.float32),
            pltpu.VMEM_SHARED((N, W48), jnp.float32),
        ],
        compiler_params=pltpu.CompilerParams(use_tc_tiling_on_sc=False),
    )(msg, dst_m, zeros_acc)


TE = 2000


def _edge_body(ea_ref, xj_ref, w1_ref, b1_ref, w2_ref,
               r_ref, s_ref, b2m_ref, c_ref, out_ref):
    h = jnp.dot(ea_ref[...].astype(jnp.bfloat16), w1_ref[...],
                preferred_element_type=jnp.float32).astype(jnp.bfloat16)
    h = h + b1_ref[...]
    h = jnp.maximum(h, jnp.bfloat16(0.01) * h)
    w = jnp.dot(h, w2_ref[...],
                preferred_element_type=jnp.float32).astype(jnp.bfloat16)
    xjb = xj_ref[...].astype(jnp.bfloat16)
    xr = jnp.dot(xjb, r_ref[...],
                 preferred_element_type=jnp.float32).astype(jnp.bfloat16)
    msg = jnp.dot(xr * w, s_ref[...], preferred_element_type=jnp.float32)
    out_ref[...] = msg + jnp.dot(xjb, b2m_ref[...],
                                 preferred_element_type=jnp.float32) + c_ref[...]


_R_EXPAND = np.zeros((D_IN, HID), dtype=np.float32)
for _i in range(D_IN):
    _R_EXPAND[_i, _i * D_OUT:(_i + 1) * D_OUT] = 1.0
_S_SELECT = np.zeros((HID, W48), dtype=np.float32)
for _i in range(D_IN):
    for _o in range(D_OUT):
        _S_SELECT[_i * D_OUT + _o, _o] = 1.0
_C_ONES = np.zeros((1, W48), dtype=np.float32)
_C_ONES[0, D_OUT] = 1.0


@jax.jit
def _tc_edge(ea, xj, W1, b1, W2, b2):
    grid = (E // TE,)
    return pl.pallas_call(
        _edge_body,
        grid=grid,
        in_specs=[
            pl.BlockSpec((TE, D_EDGE), lambda i: (i, 0)),
            pl.BlockSpec((TE, D_IN), lambda i: (i, 0)),
            pl.BlockSpec((D_EDGE, HID), lambda i: (0, 0)),
            pl.BlockSpec((1, HID), lambda i: (0, 0)),
            pl.BlockSpec((HID, HID), lambda i: (0, 0)),
            pl.BlockSpec((D_IN, HID), lambda i: (0, 0)),
            pl.BlockSpec((HID, W48), lambda i: (0, 0)),
            pl.BlockSpec((D_IN, W48), lambda i: (0, 0)),
            pl.BlockSpec((1, W48), lambda i: (0, 0)),
        ],
        out_specs=pl.BlockSpec((TE, W48), lambda i: (i, 0)),
        out_shape=jax.ShapeDtypeStruct((E, W48), jnp.float32),
        compiler_params=pltpu.CompilerParams(
            dimension_semantics=("arbitrary",)),
    )(ea, xj, W1.astype(jnp.bfloat16),
      b1.reshape(1, HID).astype(jnp.bfloat16),
      W2.astype(jnp.bfloat16),
      jnp.asarray(_R_EXPAND, dtype=jnp.bfloat16),
      jnp.asarray(_S_SELECT, dtype=jnp.bfloat16),
      jnp.pad(b2.reshape(D_IN, D_OUT),
              ((0, 0), (0, W48 - D_OUT))).astype(jnp.bfloat16),
      jnp.asarray(_C_ONES))


def _finish_body(p0_ref, p1_ref, x_ref, root_ref, cb_ref, g_ref, b_ref,
                 out_ref):
    p = p0_ref[...] + p1_ref[...]
    agg = p[:, :D_OUT] / jnp.maximum(p[:, D_OUT:D_OUT + 1], 1.0)
    pre = agg + jnp.dot(x_ref[...], root_ref[...],
                        preferred_element_type=jnp.float32) + cb_ref[...]
    mean = jnp.mean(pre, axis=0, keepdims=True)
    cen = pre - mean
    var = jnp.mean(cen * cen, axis=0, keepdims=True)
    out_ref[...] = cen * lax.rsqrt(var + 1e-5) * g_ref[...] + b_ref[...]


@jax.jit
def _tc_finish(p0, p1, x, root, conv_bias, gamma, beta):
    return pl.pallas_call(
        _finish_body,
        out_shape=jax.ShapeDtypeStruct((N, D_OUT), jnp.float32),
    )(p0, p1, x, root, conv_bias.reshape(1, D_OUT),
      gamma.reshape(1, D_OUT), beta.reshape(1, D_OUT))


def kernel(x, edge_index, edge_attr, W1, b1, W2, b2, root, conv_bias,
           gamma, beta):
    src_m = edge_index[0].astype(jnp.int32).reshape(NW * CHUNKS_W, CHUNK)
    dst_m = edge_index[1].astype(jnp.int32).reshape(NW * CHUNKS_W, CHUNK)

    xj = _sc_gather(x, src_m)
    msg = _tc_edge(edge_attr, xj, W1, b1, W2, b2)
    zeros_acc = jnp.zeros((N, W48), jnp.float32)
    p0, p1 = _sc_scatter(msg, dst_m, zeros_acc)
    return _tc_finish(p0, p1, x, root, conv_bias, gamma, beta)


# speedup vs baseline: 2.3899x; 1.0016x over previous
import jax
import jax.numpy as jnp
import numpy as np
from jax import lax
from jax.experimental import pallas as pl
from jax.experimental.pallas import tpu as pltpu
from jax.experimental.pallas import tpu_sc as plsc

N = 10000
E = 160000
D_IN = 32
D_OUT = 32
D_EDGE = 16
HID = 1024
W48 = 48

NC = 2
NS = 16
NW = NC * NS

ROWS_W = E // NW
CHUNK = 125
CHUNKS_W = ROWS_W // CHUNK
OUTER = 5
INNER = CHUNKS_W // OUTER


def _mesh():
    return plsc.VectorSubcoreMesh(core_axis_name="c", subcore_axis_name="s",
                                  num_cores=NC, num_subcores=NS)


def _gather_body(x_hbm, srcm_hbm, out_hbm, idx_v, row_v, sem):
    wid = lax.axis_index("s") * NC + lax.axis_index("c")
    pltpu.sync_copy(srcm_hbm.at[pl.ds(wid * CHUNKS_W, CHUNKS_W)], idx_v)

    def body(j, carry):
        pltpu.async_copy(x_hbm.at[idx_v.at[j]], row_v, sem).wait()
        pltpu.sync_copy(row_v,
                        out_hbm.at[pl.ds(wid * ROWS_W + j * CHUNK, CHUNK)])
        return carry

    lax.fori_loop(0, CHUNKS_W, body, 0)


@jax.jit
def _sc_gather(x, src_m):
    return pl.kernel(
        _gather_body,
        out_type=jax.ShapeDtypeStruct((E, D_IN), jnp.float32),
        mesh=_mesh(),
        scratch_types=[
            pltpu.VMEM((CHUNKS_W, CHUNK), jnp.int32),
            pltpu.VMEM((CHUNK, D_IN), jnp.float32),
            pltpu.SemaphoreType.DMA,
        ],
        compiler_params=pltpu.CompilerParams(use_tc_tiling_on_sc=False),
    )(x, src_m)


def _scatter_body(msg_hbm, dstm_hbm, zero_hbm, out0_hbm, out1_hbm,
                  idx_v, msg_v, acc_sh):
    cid = lax.axis_index("c")
    sid = lax.axis_index("s")
    wid = sid * NC + cid

    @pl.when(sid == 0)
    def _():
        pltpu.sync_copy(zero_hbm, acc_sh)

    plsc.subcore_barrier()

    pltpu.sync_copy(dstm_hbm.at[pl.ds(wid * CHUNKS_W, CHUNKS_W)], idx_v)

    def outer(c, carry):
        pltpu.sync_copy(
            msg_hbm.at[pl.ds(wid * ROWS_W + c * (INNER * CHUNK),
                             INNER * CHUNK)], msg_v)

        def inner(j, carry2):
            pltpu.sync_copy(msg_v.at[pl.ds(j * CHUNK, CHUNK)],
                            acc_sh.at[idx_v.at[c * INNER + j]], add=True)
            return carry2

        lax.fori_loop(0, INNER, inner, 0)
        return carry

    lax.fori_loop(0, OUTER, outer, 0)
    plsc.subcore_barrier()

    rows0 = sid * (N // NS)

    @pl.when(cid == 0)
    def _():
        pltpu.sync_copy(acc_sh.at[pl.ds(rows0, N // NS)],
                        out0_hbm.at[pl.ds(rows0, N // NS)])

    @pl.when(cid == 1)
    def _():
        pltpu.sync_copy(acc_sh.at[pl.ds(rows0, N // NS)],
                        out1_hbm.at[pl.ds(rows0, N // NS)])


@jax.jit
def _sc_scatter(msg, dst_m, zeros_acc):
    return pl.kernel(
        _scatter_body,
        out_type=(jax.ShapeDtypeStruct((N, W48), jnp.float32),
                  jax.ShapeDtypeStruct((N, W48), jnp.float32)),
        mesh=_mesh(),
        scratch_types=[
            pltpu.VMEM((CHUNKS_W, CHUNK), jnp.int32),
            pltpu.VMEM((INNER * CHUNK, W48), jnp.float32),
            pltpu.VMEM_SHARED((N, W48), jnp.float32),
        ],
        compiler_params=pltpu.CompilerParams(use_tc_tiling_on_sc=False),
    )(msg, dst_m, zeros_acc)


TE = 2000


def _edge_body(ea_ref, xj_ref, w1_ref, b1_ref, w2_ref,
               r_ref, s_ref, b2m_ref, c_ref, out_ref):
    h = jnp.dot(ea_ref[...].astype(jnp.bfloat16), w1_ref[...],
                preferred_element_type=jnp.float32)
    h = h + b1_ref[...]
    h = jnp.maximum(h, 0.01 * h)
    w = jnp.dot(h.astype(jnp.bfloat16), w2_ref[...],
                preferred_element_type=jnp.float32)
    xjb = xj_ref[...].astype(jnp.bfloat16)
    xr = jnp.dot(xjb, r_ref[...],
                 preferred_element_type=jnp.float32)
    msg = jnp.dot((xr * w).astype(jnp.bfloat16), s_ref[...],
                  preferred_element_type=jnp.float32)
    out_ref[...] = msg + jnp.dot(xjb, b2m_ref[...],
                                 preferred_element_type=jnp.float32) + c_ref[...]


_R_EXPAND = np.zeros((D_IN, HID), dtype=np.float32)
for _i in range(D_IN):
    _R_EXPAND[_i, _i * D_OUT:(_i + 1) * D_OUT] = 1.0
_S_SELECT = np.zeros((HID, W48), dtype=np.float32)
for _i in range(D_IN):
    for _o in range(D_OUT):
        _S_SELECT[_i * D_OUT + _o, _o] = 1.0
_C_ONES = np.zeros((1, W48), dtype=np.float32)
_C_ONES[0, D_OUT] = 1.0


@jax.jit
def _tc_edge(ea, xj, W1, b1, W2, b2):
    grid = (E // TE,)
    return pl.pallas_call(
        _edge_body,
        grid=grid,
        in_specs=[
            pl.BlockSpec((TE, D_EDGE), lambda i: (i, 0)),
            pl.BlockSpec((TE, D_IN), lambda i: (i, 0)),
            pl.BlockSpec((D_EDGE, HID), lambda i: (0, 0)),
            pl.BlockSpec((1, HID), lambda i: (0, 0)),
            pl.BlockSpec((HID, HID), lambda i: (0, 0)),
            pl.BlockSpec((D_IN, HID), lambda i: (0, 0)),
            pl.BlockSpec((HID, W48), lambda i: (0, 0)),
            pl.BlockSpec((D_IN, W48), lambda i: (0, 0)),
            pl.BlockSpec((1, W48), lambda i: (0, 0)),
        ],
        out_specs=pl.BlockSpec((TE, W48), lambda i: (i, 0)),
        out_shape=jax.ShapeDtypeStruct((E, W48), jnp.float32),
        compiler_params=pltpu.CompilerParams(
            dimension_semantics=("arbitrary",)),
    )(ea, xj, W1.astype(jnp.bfloat16),
      b1.reshape(1, HID),
      W2.astype(jnp.bfloat16),
      jnp.asarray(_R_EXPAND, dtype=jnp.bfloat16),
      jnp.asarray(_S_SELECT, dtype=jnp.bfloat16),
      jnp.pad(b2.reshape(D_IN, D_OUT),
              ((0, 0), (0, W48 - D_OUT))).astype(jnp.bfloat16),
      jnp.asarray(_C_ONES))


def _finish_body(p0_ref, p1_ref, x_ref, root_ref, cb_ref, g_ref, b_ref,
                 out_ref):
    p = p0_ref[...] + p1_ref[...]
    agg = p[:, :D_OUT] / jnp.maximum(p[:, D_OUT:D_OUT + 1], 1.0)
    pre = agg + jnp.dot(x_ref[...], root_ref[...],
                        preferred_element_type=jnp.float32) + cb_ref[...]
    mean = jnp.mean(pre, axis=0, keepdims=True)
    cen = pre - mean
    var = jnp.mean(cen * cen, axis=0, keepdims=True)
    out_ref[...] = cen * lax.rsqrt(var + 1e-5) * g_ref[...] + b_ref[...]


@jax.jit
def _tc_finish(p0, p1, x, root, conv_bias, gamma, beta):
    return pl.pallas_call(
        _finish_body,
        out_shape=jax.ShapeDtypeStruct((N, D_OUT), jnp.float32),
    )(p0, p1, x, root, conv_bias.reshape(1, D_OUT),
      gamma.reshape(1, D_OUT), beta.reshape(1, D_OUT))


def kernel(x, edge_index, edge_attr, W1, b1, W2, b2, root, conv_bias,
           gamma, beta):
    src_m = edge_index[0].astype(jnp.int32).reshape(NW * CHUNKS_W, CHUNK)
    dst_m = edge_index[1].astype(jnp.int32).reshape(NW * CHUNKS_W, CHUNK)

    xj = _sc_gather(x, src_m)
    msg = _tc_edge(edge_attr, xj, W1, b1, W2, b2)
    zeros_acc = jnp.zeros((N, W48), jnp.float32)
    p0, p1 = _sc_scatter(msg, dst_m, zeros_acc)
    return _tc_finish(p0, p1, x, root, conv_bias, gamma, beta)
